# Initial kernel scaffold; baseline (speedup 1.0000x reference)
import jax
import jax.numpy as jnp
from jax.experimental import pallas as pl


def kernel(node_feat, edge_index, W0, b0, g0, be0, W1, b1, g1, be1):
    raise NotImplementedError("write your pallas kernel here")


# speedup vs baseline: 8.4265x; 8.4265x over previous
import functools

import jax
import jax.numpy as jnp
from jax import lax
from jax.experimental import pallas as pl
from jax.experimental.pallas import tpu as pltpu
from jax.experimental.pallas import tpu_sc as plsc

_N = 10000
_E = 320000
_D = 128
_H = 256
_HH = _H // 2
_NT = 16
_NP = 10240
_RPT = _NP // _NT
_K = 80
_RB = 128

_mesh = plsc.VectorSubcoreMesh(core_axis_name="c", subcore_axis_name="s")


def _fill_iota(idxbuf, base):
    @pl.loop(0, _RB // 16)
    def _(k):
        idxbuf[pl.ds(k * 16, 16)] = base + k * 16 + lax.iota(jnp.int32, 16)


@functools.partial(
    pl.kernel,
    out_type=jax.ShapeDtypeStruct((2 * _NP, 16), jnp.float32),
    mesh=_mesh,
    scratch_types=[
        pltpu.VMEM((_K,), jnp.int32),
        pltpu.VMEM((_K, 16), jnp.float32),
        pltpu.VMEM((_RB, 16), jnp.float32),
        pltpu.VMEM((_RB,), jnp.int32),
        pltpu.VMEM_SHARED((_NP, 16), jnp.float32),
        pltpu.SemaphoreType.DMA,
    ],
)
def _deg_kernel(dst_ref, out_ref, didx, ones_buf, zbuf, idxbuf, deg_sh, sem):
    c = lax.axis_index("c")
    s = lax.axis_index("s")

    @pl.loop(0, _K)
    def _(i):
        ones_buf[i, :] = jnp.ones((16,), jnp.float32)

    @pl.loop(0, _RB)
    def _(i):
        zbuf[i, :] = jnp.zeros((16,), jnp.float32)

    @pl.loop(0, _RPT // _RB)
    def _(j):
        base = s * _RPT + j * _RB
        _fill_iota(idxbuf, base)
        pltpu.sync_copy(zbuf, deg_sh.at[idxbuf])

    plsc.subcore_barrier()

    ept = _E // 32
    base_e = (c * _NT + s) * ept

    @pl.loop(0, ept // _K)
    def _(i):
        pltpu.sync_copy(dst_ref.at[pl.ds(base_e + i * _K, _K)], didx)
        pltpu.sync_copy(ones_buf, deg_sh.at[didx], add=True)

    plsc.subcore_barrier()

    @pl.loop(0, _RPT // _RB)
    def _(j):
        base = s * _RPT + j * _RB
        _fill_iota(idxbuf, base)
        pltpu.async_copy(deg_sh.at[idxbuf], zbuf, sem).wait()
        pltpu.sync_copy(zbuf, out_ref.at[pl.ds(c * _NP + base, _RB)])


@functools.partial(
    pl.kernel,
    out_type=jax.ShapeDtypeStruct((2 * _NP, _HH), jnp.float32),
    mesh=_mesh,
    scratch_types=[
        pltpu.VMEM((_K,), jnp.int32),
        pltpu.VMEM((_K,), jnp.int32),
        pltpu.VMEM((_K, _HH), jnp.float32),
        pltpu.VMEM((_RB, _HH), jnp.float32),
        pltpu.VMEM((_RB,), jnp.int32),
        pltpu.VMEM_SHARED((_NP, _HH), jnp.float32),
        pltpu.SemaphoreType.DMA,
    ],
)
def _seg_kernel(ht_ref, src_ref, dst_ref, out_ref, sidx, didx, rows, bounce,
                idxbuf, acc_sh, gsem):
    c = lax.axis_index("c")
    s = lax.axis_index("s")
    ept = _E // _NT
    nch = ept // _K
    off = c * _NP

    @pl.loop(0, _RPT // _RB)
    def _(j):
        base = s * _RPT + j * _RB
        _fill_iota(idxbuf, base)
        pltpu.sync_copy(ht_ref.at[pl.ds(off + base, _RB)], bounce)
        pltpu.sync_copy(bounce, acc_sh.at[idxbuf])

    plsc.subcore_barrier()

    @pl.loop(0, nch)
    def _(i):
        base = s * ept + i * _K
        pltpu.sync_copy(src_ref.at[pl.ds(base, _K)], sidx)
        pltpu.sync_copy(dst_ref.at[pl.ds(base, _K)], didx)

        @pl.loop(0, _K // 16)
        def _(j):
            sidx[pl.ds(j * 16, 16)] = sidx[pl.ds(j * 16, 16)] + off

        pltpu.async_copy(ht_ref.at[sidx], rows, gsem).wait()
        pltpu.sync_copy(rows, acc_sh.at[didx], add=True)

    plsc.subcore_barrier()

    @pl.loop(0, _RPT // _RB)
    def _(j):
        base = s * _RPT + j * _RB
        _fill_iota(idxbuf, base)
        pltpu.async_copy(acc_sh.at[idxbuf], bounce, gsem).wait()
        pltpu.sync_copy(bounce, out_ref.at[pl.ds(off + base, _RB)])


def _tc1_body(deg_ref, x_ref, w_ref, b_ref, ht_ref, dis_ref):
    deg = 1.0 + deg_ref[0:_N, 0:1] + deg_ref[_NP:_NP + _N, 0:1]
    dis = lax.rsqrt(deg)
    dis_ref[...] = dis
    h = jnp.dot(x_ref[...], w_ref[...], preferred_element_type=jnp.float32)
    ht = (h + b_ref[...]) * dis
    ht_ref[0:_N, :] = ht[:, 0:_HH]
    ht_ref[_NP:_NP + _N, :] = ht[:, _HH:_H]


def _tc2_body(acc_ref, dis_ref, g_ref, be_ref, w_ref, b_ref, ht_ref):
    dis = dis_ref[...]
    agg = jnp.concatenate([acc_ref[0:_N, :], acc_ref[_NP:_NP + _N, :]],
                          axis=1) * dis
    mu = jnp.mean(agg, axis=0, keepdims=True)
    var = jnp.mean((agg - mu) * (agg - mu), axis=0, keepdims=True)
    xn = jnp.maximum(
        g_ref[...] * (agg - mu) * lax.rsqrt(var + 1e-5) + be_ref[...], 0.0)
    h = jnp.dot(xn, w_ref[...], preferred_element_type=jnp.float32)
    ht = (h + b_ref[...]) * dis
    ht_ref[0:_N, :] = ht[:, 0:_HH]
    ht_ref[_NP:_NP + _N, :] = ht[:, _HH:_H]


def _tc3_body(acc_ref, dis_ref, g_ref, be_ref, out_ref):
    agg = jnp.concatenate([acc_ref[0:_N, :], acc_ref[_NP:_NP + _N, :]],
                          axis=1) * dis_ref[...]
    mu = jnp.mean(agg, axis=0, keepdims=True)
    var = jnp.mean((agg - mu) * (agg - mu), axis=0, keepdims=True)
    out_ref[...] = jnp.maximum(
        g_ref[...] * (agg - mu) * lax.rsqrt(var + 1e-5) + be_ref[...], 0.0)


_tc1 = pl.pallas_call(
    _tc1_body,
    out_shape=(jax.ShapeDtypeStruct((2 * _NP, _HH), jnp.float32),
               jax.ShapeDtypeStruct((_N, 1), jnp.float32)))

_tc2 = pl.pallas_call(
    _tc2_body,
    out_shape=jax.ShapeDtypeStruct((2 * _NP, _HH), jnp.float32))

_tc3 = pl.pallas_call(
    _tc3_body,
    out_shape=jax.ShapeDtypeStruct((_N, _H), jnp.float32))


def kernel(node_feat, edge_index, W0, b0, g0, be0, W1, b1, g1, be1):
    src = edge_index[0]
    dst = edge_index[1]
    deg2 = _deg_kernel(dst)
    ht0, dis = _tc1(deg2, node_feat, W0, b0.reshape(1, _H))
    acc0 = _seg_kernel(ht0, src, dst)
    ht1 = _tc2(acc0, dis, g0.reshape(1, _H), be0.reshape(1, _H), W1,
               b1.reshape(1, _H))
    acc1 = _seg_kernel(ht1, src, dst)
    return _tc3(acc1, dis, g1.reshape(1, _H), be1.reshape(1, _H))


# speedup vs baseline: 13.3929x; 1.5894x over previous
import functools

import jax
import jax.numpy as jnp
from jax import lax
from jax.experimental import pallas as pl
from jax.experimental.pallas import tpu as pltpu
from jax.experimental.pallas import tpu_sc as plsc

_N = 10000
_E = 320000
_D = 128
_H = 256
_HH = _H // 2
_NT = 16
_NP = 10240
_RPT = _NP // _NT
_K = 80
_RB = 128

_mesh = plsc.VectorSubcoreMesh(core_axis_name="c", subcore_axis_name="s")


def _fill_iota(idxbuf, base):
    @pl.loop(0, _RB // 16)
    def _(k):
        idxbuf[pl.ds(k * 16, 16)] = base + k * 16 + lax.iota(jnp.int32, 16)


@functools.partial(
    pl.kernel,
    out_type=jax.ShapeDtypeStruct((2 * _NP, 16), jnp.float32),
    mesh=_mesh,
    scratch_types=[
        pltpu.VMEM((_K,), jnp.int32),
        pltpu.VMEM((_K, 16), jnp.float32),
        pltpu.VMEM((_RB, 16), jnp.float32),
        pltpu.VMEM((_RB,), jnp.int32),
        pltpu.VMEM_SHARED((_NP, 16), jnp.float32),
        pltpu.SemaphoreType.DMA,
    ],
)
def _deg_kernel(dst_ref, out_ref, didx, ones_buf, zbuf, idxbuf, deg_sh, sem):
    c = lax.axis_index("c")
    s = lax.axis_index("s")

    @pl.loop(0, _K)
    def _(i):
        ones_buf[i, :] = jnp.ones((16,), jnp.float32)

    @pl.loop(0, _RB)
    def _(i):
        zbuf[i, :] = jnp.zeros((16,), jnp.float32)

    @pl.loop(0, _RPT // _RB)
    def _(j):
        base = s * _RPT + j * _RB
        _fill_iota(idxbuf, base)
        pltpu.sync_copy(zbuf, deg_sh.at[idxbuf])

    plsc.subcore_barrier()

    ept = _E // 32
    base_e = (c * _NT + s) * ept

    @pl.loop(0, ept // _K)
    def _(i):
        pltpu.sync_copy(dst_ref.at[pl.ds(base_e + i * _K, _K)], didx)
        pltpu.sync_copy(ones_buf, deg_sh.at[didx], add=True)

    plsc.subcore_barrier()

    @pl.loop(0, _RPT // _RB)
    def _(j):
        base = s * _RPT + j * _RB
        _fill_iota(idxbuf, base)
        pltpu.async_copy(deg_sh.at[idxbuf], zbuf, sem).wait()
        pltpu.sync_copy(zbuf, out_ref.at[pl.ds(c * _NP + base, _RB)])


@functools.partial(
    pl.kernel,
    out_type=jax.ShapeDtypeStruct((2 * _NP, _HH), jnp.float32),
    mesh=_mesh,
    scratch_types=[
        pltpu.VMEM((_K,), jnp.int32),
        pltpu.VMEM((_K,), jnp.int32),
        pltpu.VMEM((_K,), jnp.int32),
        pltpu.VMEM((_K,), jnp.int32),
        pltpu.VMEM((_K, _HH), jnp.float32),
        pltpu.VMEM((_K, _HH), jnp.float32),
        pltpu.VMEM((_RB, _HH), jnp.float32),
        pltpu.VMEM((_RB,), jnp.int32),
        pltpu.VMEM_SHARED((_NP, _HH), jnp.float32),
        pltpu.SemaphoreType.DMA,
        pltpu.SemaphoreType.DMA,
    ],
)
def _seg_kernel(ht_ref, src_ref, dst_ref, out_ref, sidx0, sidx1, didx0, didx1,
                rows0, rows1, bounce, idxbuf, acc_sh, gsem0, gsem1):
    c = lax.axis_index("c")
    s = lax.axis_index("s")
    ept = _E // _NT
    nch = ept // _K
    off = c * _NP

    @pl.loop(0, _RPT // _RB)
    def _(j):
        base = s * _RPT + j * _RB
        _fill_iota(idxbuf, base)
        pltpu.sync_copy(ht_ref.at[pl.ds(off + base, _RB)], bounce)
        pltpu.sync_copy(bounce, acc_sh.at[idxbuf])

    plsc.subcore_barrier()

    def _start_gather(i, sidx, didx, rows, gsem):
        base = s * ept + i * _K
        pltpu.sync_copy(src_ref.at[pl.ds(base, _K)], sidx)
        pltpu.sync_copy(dst_ref.at[pl.ds(base, _K)], didx)

        @pl.loop(0, _K // 16)
        def _(j):
            sidx[pl.ds(j * 16, 16)] = sidx[pl.ds(j * 16, 16)] + off

        return pltpu.async_copy(ht_ref.at[sidx], rows, gsem)

    _start_gather(0, sidx0, didx0, rows0, gsem0)

    @pl.loop(0, nch // 2)
    def _(j):
        c0 = 2 * j
        _start_gather(c0 + 1, sidx1, didx1, rows1, gsem1)
        pltpu.make_async_copy(ht_ref.at[sidx0], rows0, gsem0).wait()
        pltpu.sync_copy(rows0, acc_sh.at[didx0], add=True)

        @pl.when(j < nch // 2 - 1)
        def _():
            _start_gather(c0 + 2, sidx0, didx0, rows0, gsem0)

        pltpu.make_async_copy(ht_ref.at[sidx1], rows1, gsem1).wait()
        pltpu.sync_copy(rows1, acc_sh.at[didx1], add=True)

    plsc.subcore_barrier()

    @pl.loop(0, _RPT // _RB)
    def _(j):
        base = s * _RPT + j * _RB
        _fill_iota(idxbuf, base)
        pltpu.async_copy(acc_sh.at[idxbuf], bounce, gsem0).wait()
        pltpu.sync_copy(bounce, out_ref.at[pl.ds(off + base, _RB)])


def _tc1_body(deg_ref, x_ref, w_ref, b_ref, ht_ref, dis_ref):
    deg = 1.0 + deg_ref[0:_N, 0:1] + deg_ref[_NP:_NP + _N, 0:1]
    dis = lax.rsqrt(deg)
    dis_ref[...] = dis
    h = jnp.dot(x_ref[...], w_ref[...], preferred_element_type=jnp.float32)
    ht = (h + b_ref[...]) * dis
    ht_ref[0:_N, :] = ht[:, 0:_HH]
    ht_ref[_NP:_NP + _N, :] = ht[:, _HH:_H]


def _tc2_body(acc_ref, dis_ref, g_ref, be_ref, w_ref, b_ref, ht_ref):
    dis = dis_ref[...]
    agg = jnp.concatenate([acc_ref[0:_N, :], acc_ref[_NP:_NP + _N, :]],
                          axis=1) * dis
    mu = jnp.mean(agg, axis=0, keepdims=True)
    var = jnp.mean((agg - mu) * (agg - mu), axis=0, keepdims=True)
    xn = jnp.maximum(
        g_ref[...] * (agg - mu) * lax.rsqrt(var + 1e-5) + be_ref[...], 0.0)
    h = jnp.dot(xn, w_ref[...], preferred_element_type=jnp.float32)
    ht = (h + b_ref[...]) * dis
    ht_ref[0:_N, :] = ht[:, 0:_HH]
    ht_ref[_NP:_NP + _N, :] = ht[:, _HH:_H]


def _tc3_body(acc_ref, dis_ref, g_ref, be_ref, out_ref):
    agg = jnp.concatenate([acc_ref[0:_N, :], acc_ref[_NP:_NP + _N, :]],
                          axis=1) * dis_ref[...]
    mu = jnp.mean(agg, axis=0, keepdims=True)
    var = jnp.mean((agg - mu) * (agg - mu), axis=0, keepdims=True)
    out_ref[...] = jnp.maximum(
        g_ref[...] * (agg - mu) * lax.rsqrt(var + 1e-5) + be_ref[...], 0.0)


_tc1 = pl.pallas_call(
    _tc1_body,
    out_shape=(jax.ShapeDtypeStruct((2 * _NP, _HH), jnp.float32),
               jax.ShapeDtypeStruct((_N, 1), jnp.float32)))

_tc2 = pl.pallas_call(
    _tc2_body,
    out_shape=jax.ShapeDtypeStruct((2 * _NP, _HH), jnp.float32))

_tc3 = pl.pallas_call(
    _tc3_body,
    out_shape=jax.ShapeDtypeStruct((_N, _H), jnp.float32))


def kernel(node_feat, edge_index, W0, b0, g0, be0, W1, b1, g1, be1):
    src = edge_index[0]
    dst = edge_index[1]
    deg2 = _deg_kernel(dst)
    ht0, dis = _tc1(deg2, node_feat, W0, b0.reshape(1, _H))
    acc0 = _seg_kernel(ht0, src, dst)
    ht1 = _tc2(acc0, dis, g0.reshape(1, _H), be0.reshape(1, _H), W1,
               b1.reshape(1, _H))
    acc1 = _seg_kernel(ht1, src, dst)
    return _tc3(acc1, dis, g1.reshape(1, _H), be1.reshape(1, _H))


# speedup vs baseline: 15.8430x; 1.1829x over previous
import functools

import jax
import jax.numpy as jnp
from jax import lax
from jax.experimental import pallas as pl
from jax.experimental.pallas import tpu as pltpu
from jax.experimental.pallas import tpu_sc as plsc

_N = 10000
_E = 320000
_D = 128
_H = 256
_HH = _H // 2
_NT = 16
_NP = 10240
_RPT = _NP // _NT
_K = 80
_KE = 128
_RB = 128
_RBS = 64
_KT = 32

_mesh = plsc.VectorSubcoreMesh(core_axis_name="c", subcore_axis_name="s")


def _fill_iota(idxbuf, base, n):
    @pl.loop(0, n // 16)
    def _(k):
        idxbuf[pl.ds(k * 16, 16)] = base + k * 16 + lax.iota(jnp.int32, 16)


@functools.partial(
    pl.kernel,
    out_type=jax.ShapeDtypeStruct((2 * _NP, 16), jnp.float32),
    mesh=_mesh,
    scratch_types=[
        pltpu.VMEM((_K,), jnp.int32),
        pltpu.VMEM((_K, 16), jnp.float32),
        pltpu.VMEM((_RB, 16), jnp.float32),
        pltpu.VMEM((_RB,), jnp.int32),
        pltpu.VMEM_SHARED((_NP, 16), jnp.float32),
        pltpu.SemaphoreType.DMA,
    ],
)
def _deg_kernel(dst_ref, out_ref, didx, ones_buf, zbuf, idxbuf, deg_sh, sem):
    c = lax.axis_index("c")
    s = lax.axis_index("s")

    @pl.loop(0, _K)
    def _(i):
        ones_buf[i, :] = jnp.ones((16,), jnp.float32)

    @pl.loop(0, _RB)
    def _(i):
        zbuf[i, :] = jnp.zeros((16,), jnp.float32)

    @pl.loop(0, _RPT // _RB)
    def _(j):
        base = s * _RPT + j * _RB
        _fill_iota(idxbuf, base, _RB)
        pltpu.sync_copy(zbuf, deg_sh.at[idxbuf])

    plsc.subcore_barrier()

    ept = _E // 32
    base_e = (c * _NT + s) * ept

    @pl.loop(0, ept // _K)
    def _(i):
        pltpu.sync_copy(dst_ref.at[pl.ds(base_e + i * _K, _K)], didx)
        pltpu.sync_copy(ones_buf, deg_sh.at[didx], add=True)

    plsc.subcore_barrier()

    @pl.loop(0, _RPT // _RB)
    def _(j):
        base = s * _RPT + j * _RB
        _fill_iota(idxbuf, base, _RB)
        pltpu.async_copy(deg_sh.at[idxbuf], zbuf, sem).wait()
        pltpu.sync_copy(zbuf, out_ref.at[pl.ds(c * _NP + base, _RB)])


@functools.partial(
    pl.kernel,
    out_type=jax.ShapeDtypeStruct((2 * _NP, _HH), jnp.float32),
    mesh=_mesh,
    scratch_types=[
        pltpu.VMEM((_KE,), jnp.int32),
        pltpu.VMEM((_KE,), jnp.int32),
        pltpu.VMEM((_KE,), jnp.int32),
        pltpu.VMEM((_KE,), jnp.int32),
        pltpu.VMEM((_KE, _HH), jnp.float32),
        pltpu.VMEM((_KE, _HH), jnp.float32),
        pltpu.VMEM((_KT,), jnp.int32),
        pltpu.VMEM((_KT,), jnp.int32),
        pltpu.VMEM((_KT, _HH), jnp.float32),
        pltpu.VMEM((_RBS, _HH), jnp.float32),
        pltpu.VMEM((_RBS,), jnp.int32),
        pltpu.VMEM_SHARED((_NP, _HH), jnp.float32),
        pltpu.SemaphoreType.DMA,
        pltpu.SemaphoreType.DMA,
    ],
)
def _seg_kernel(ht_ref, src_ref, dst_ref, out_ref, sidx0, sidx1, didx0, didx1,
                rows0, rows1, sidxT, didxT, rowsT, bounce, idxbuf, acc_sh,
                gsem0, gsem1):
    c = lax.axis_index("c")
    s = lax.axis_index("s")
    ept = _E // _NT
    nfull = ept // _KE
    tail = ept - nfull * _KE
    off = c * _NP

    @pl.loop(0, _RPT // _RBS)
    def _(j):
        base = s * _RPT + j * _RBS
        _fill_iota(idxbuf, base, _RBS)
        pltpu.sync_copy(ht_ref.at[pl.ds(off + base, _RBS)], bounce)
        pltpu.sync_copy(bounce, acc_sh.at[idxbuf])

    plsc.subcore_barrier()

    def _start_gather(base, n, sidx, didx, rows, gsem):
        pltpu.sync_copy(src_ref.at[pl.ds(base, n)], sidx)
        pltpu.sync_copy(dst_ref.at[pl.ds(base, n)], didx)

        @pl.loop(0, n // 16)
        def _(j):
            sidx[pl.ds(j * 16, 16)] = sidx[pl.ds(j * 16, 16)] + off

        return pltpu.async_copy(ht_ref.at[sidx], rows, gsem)

    def _finish(sidx, didx, rows, gsem):
        pltpu.make_async_copy(ht_ref.at[sidx], rows, gsem).wait()
        pltpu.sync_copy(rows, acc_sh.at[didx], add=True)

    e0 = s * ept
    _start_gather(e0, _KE, sidx0, didx0, rows0, gsem0)

    @pl.loop(0, nfull // 2)
    def _(j):
        c0 = 2 * j
        _start_gather(e0 + (c0 + 1) * _KE, _KE, sidx1, didx1, rows1, gsem1)
        _finish(sidx0, didx0, rows0, gsem0)

        @pl.when(j < nfull // 2 - 1)
        def _():
            _start_gather(e0 + (c0 + 2) * _KE, _KE, sidx0, didx0, rows0,
                          gsem0)

        _finish(sidx1, didx1, rows1, gsem1)

    _start_gather(e0 + nfull * _KE, tail, sidxT, didxT, rowsT, gsem0)
    _finish(sidxT, didxT, rowsT, gsem0)

    plsc.subcore_barrier()

    @pl.loop(0, _RPT // _RBS)
    def _(j):
        base = s * _RPT + j * _RBS
        _fill_iota(idxbuf, base, _RBS)
        pltpu.async_copy(acc_sh.at[idxbuf], bounce, gsem0).wait()
        pltpu.sync_copy(bounce, out_ref.at[pl.ds(off + base, _RBS)])


def _tc1_body(deg_ref, x_ref, w_ref, b_ref, ht_ref, dis_ref):
    deg = 1.0 + deg_ref[0:_N, 0:1] + deg_ref[_NP:_NP + _N, 0:1]
    dis = lax.rsqrt(deg)
    dis_ref[...] = dis
    h = jnp.dot(x_ref[...], w_ref[...], preferred_element_type=jnp.float32)
    ht = (h + b_ref[...]) * dis
    ht_ref[0:_N, :] = ht[:, 0:_HH]
    ht_ref[_NP:_NP + _N, :] = ht[:, _HH:_H]


def _tc2_body(acc_ref, dis_ref, g_ref, be_ref, w_ref, b_ref, ht_ref):
    dis = dis_ref[...]
    agg = jnp.concatenate([acc_ref[0:_N, :], acc_ref[_NP:_NP + _N, :]],
                          axis=1) * dis
    mu = jnp.mean(agg, axis=0, keepdims=True)
    var = jnp.mean((agg - mu) * (agg - mu), axis=0, keepdims=True)
    xn = jnp.maximum(
        g_ref[...] * (agg - mu) * lax.rsqrt(var + 1e-5) + be_ref[...], 0.0)
    h = jnp.dot(xn, w_ref[...], preferred_element_type=jnp.float32)
    ht = (h + b_ref[...]) * dis
    ht_ref[0:_N, :] = ht[:, 0:_HH]
    ht_ref[_NP:_NP + _N, :] = ht[:, _HH:_H]


def _tc3_body(acc_ref, dis_ref, g_ref, be_ref, out_ref):
    agg = jnp.concatenate([acc_ref[0:_N, :], acc_ref[_NP:_NP + _N, :]],
                          axis=1) * dis_ref[...]
    mu = jnp.mean(agg, axis=0, keepdims=True)
    var = jnp.mean((agg - mu) * (agg - mu), axis=0, keepdims=True)
    out_ref[...] = jnp.maximum(
        g_ref[...] * (agg - mu) * lax.rsqrt(var + 1e-5) + be_ref[...], 0.0)


_tc1 = pl.pallas_call(
    _tc1_body,
    out_shape=(jax.ShapeDtypeStruct((2 * _NP, _HH), jnp.float32),
               jax.ShapeDtypeStruct((_N, 1), jnp.float32)))

_tc2 = pl.pallas_call(
    _tc2_body,
    out_shape=jax.ShapeDtypeStruct((2 * _NP, _HH), jnp.float32))

_tc3 = pl.pallas_call(
    _tc3_body,
    out_shape=jax.ShapeDtypeStruct((_N, _H), jnp.float32))


def kernel(node_feat, edge_index, W0, b0, g0, be0, W1, b1, g1, be1):
    src = edge_index[0]
    dst = edge_index[1]
    deg2 = _deg_kernel(dst)
    ht0, dis = _tc1(deg2, node_feat, W0, b0.reshape(1, _H))
    acc0 = _seg_kernel(ht0, src, dst)
    ht1 = _tc2(acc0, dis, g0.reshape(1, _H), be0.reshape(1, _H), W1,
               b1.reshape(1, _H))
    acc1 = _seg_kernel(ht1, src, dst)
    return _tc3(acc1, dis, g1.reshape(1, _H), be1.reshape(1, _H))
